# baseline (speedup 1.0000x reference)
import functools

import jax
import jax.numpy as jnp
from jax import lax
from jax.experimental import pallas as pl
from jax.experimental.pallas import tpu as pltpu
from jax.experimental.pallas import tpu_sc as plsc

_NC = 2
_NS = 16
_NW = _NC * _NS

_B = 1024
_S = 200
_D = 64
_ROWS = _B * _S
_RPW = _ROWS // _NW
_IDXW = 128
_NSTREAM = _RPW // _IDXW
_CH_STREAMS = 10
_CH_ROWS = _CH_STREAMS * _IDXW
_NCH = _NSTREAM // _CH_STREAMS

_EPS = 1e-5


def _sc_gather_t(table, idx3d):
    mesh = plsc.VectorSubcoreMesh(core_axis_name="c", subcore_axis_name="s")

    @functools.partial(
        pl.kernel,
        mesh=mesh,
        compiler_params=pltpu.CompilerParams(use_tc_tiling_on_sc=False,
                                             needs_layout_passes=False),
        out_type=jax.ShapeDtypeStruct((_S, _D // 8, _B // 128, 8 * 128),
                                      jnp.float32),
        scratch_types=[
            pltpu.VMEM((_NSTREAM, _IDXW), jnp.int32),
            pltpu.VMEM((_IDXW, _D), jnp.float32),
            pltpu.VMEM((_D // 8, 8 * 128), jnp.float32),
            pltpu.SemaphoreType.DMA,
        ],
    )
    def k(tab_hbm, idx_hbm, out_hbm, idx_v, rows_v, t_v, sem):
        wid = lax.axis_index("s") * _NC + lax.axis_index("c")
        pltpu.sync_copy(idx_hbm.at[wid], idx_v)
        lane = lax.iota(jnp.int32, 16)
        row_idx = [lane + g * 16 for g in range(8)]

        def body(j, carry):
            pltpu.async_copy(tab_hbm.at[idx_v.at[j]], rows_v, sem).wait()
            for d in range(_D):
                dcol = jnp.full((16,), d, jnp.int32)
                dst = (d % 8) * 128
                for g in range(8):
                    t_v[d // 8, pl.ds(dst + g * 16, 16)] = plsc.load_gather(
                        rows_v, [row_idx[g], dcol])
            blk = wid * _NSTREAM + j
            pltpu.sync_copy(t_v, out_hbm.at[blk // 8, :, blk % 8, :])
            return carry

        lax.fori_loop(0, _NSTREAM, body, 0)

    return k(table, idx3d)


_TCOL = 4096
_TGRID = 124
_SPLIT = _TGRID * _TCOL
_VLAST = -(-1000000 // _TCOL) - 1


def _tr_block(xa_ref, xb_ref, o_ref):
    o_ref[:, :_D] = jnp.swapaxes(xa_ref[...], 0, 1)
    o_ref[:, _D:] = jnp.swapaxes(xb_ref[...], 0, 1)


def _tc_transpose(table_t):
    return pl.pallas_call(
        _tr_block,
        grid=(_TGRID,),
        in_specs=[
            pl.BlockSpec((_D, _TCOL), lambda i: (0, i)),
            pl.BlockSpec((_D, _TCOL),
                         lambda i: (0, jnp.minimum(i + _TGRID, _VLAST))),
        ],
        out_specs=pl.BlockSpec((_TCOL, 128), lambda i: (i, 0)),
        out_shape=jax.ShapeDtypeStruct((_SPLIT, 128), jnp.float32),
    )(table_t, table_t)


_SB = 8


def _ln_block(x_ref, pe_ref, g_ref, b_ref, o_ref):
    x = x_ref[...] + pe_ref[...]
    m = jnp.mean(x, axis=1, keepdims=True)
    c = x - m
    v = jnp.mean(c * c, axis=1, keepdims=True)
    y = c * lax.rsqrt(v + _EPS)
    o_ref[...] = y * g_ref[...] + b_ref[...]


def _tc_layernorm(xt, pe_t, gamma_t, beta_t):
    grid = _S // _SB
    return pl.pallas_call(
        _ln_block,
        grid=(grid,),
        in_specs=[
            pl.BlockSpec((_SB, _D, _B), lambda i: (i, 0, 0)),
            pl.BlockSpec((_SB, _D, 1), lambda i: (i, 0, 0)),
            pl.BlockSpec((1, _D, 1), lambda i: (0, 0, 0)),
            pl.BlockSpec((1, _D, 1), lambda i: (0, 0, 0)),
        ],
        out_specs=pl.BlockSpec((_SB, _D, _B), lambda i: (i, 0, 0)),
        out_shape=jax.ShapeDtypeStruct((_S, _D, _B), jnp.float32),
    )(xt, pe_t, gamma_t, beta_t)


def kernel(input_ids, table, gamma, beta, pos_enc):
    ids = input_ids.T.reshape(-1).astype(jnp.int32)
    gidx = jnp.where(ids < _SPLIT, ids * 2, (ids - _SPLIT) * 2 + 1)
    idx3d = gidx.reshape(_NW, _NSTREAM, _IDXW)
    lines = _tc_transpose(table.T)
    tab_lin = lines.reshape(-1).reshape(2 * _SPLIT, _D)
    st5 = _sc_gather_t(tab_lin, idx3d)
    xt = (st5.reshape(_S, 8, 8, 8, 128).transpose(0, 1, 3, 2, 4)
          .reshape(_S, _D, _B))
    pe_t = pos_enc[0, :_S, :].reshape(_S, _D, 1)
    out_t = _tc_layernorm(xt, pe_t,
                          gamma.reshape(1, _D, 1), beta.reshape(1, _D, 1))
    return out_t.transpose(2, 0, 1)


# speedup vs baseline: 1.5983x; 1.5983x over previous
import functools

import jax
import jax.numpy as jnp
from jax import lax
from jax.experimental import pallas as pl
from jax.experimental.pallas import tpu as pltpu
from jax.experimental.pallas import tpu_sc as plsc

_NC = 2
_NS = 16
_NW = _NC * _NS

_B = 1024
_S = 200
_D = 64
_ROWS = _B * _S
_RPW = _ROWS // _NW
_IDXW = 128
_NSTREAM = _RPW // _IDXW
_CH_STREAMS = 10
_CH_ROWS = _CH_STREAMS * _IDXW
_NCH = _NSTREAM // _CH_STREAMS

_EPS = 1e-5


def _sc_gather(table, idx3d):
    mesh = plsc.VectorSubcoreMesh(core_axis_name="c", subcore_axis_name="s")

    @functools.partial(
        pl.kernel,
        mesh=mesh,
        compiler_params=pltpu.CompilerParams(use_tc_tiling_on_sc=False),
        out_type=jax.ShapeDtypeStruct((_ROWS, _D), jnp.float32),
        scratch_types=[
            pltpu.VMEM((_NSTREAM, _IDXW), jnp.int32),
            pltpu.VMEM((_CH_ROWS, _D), jnp.float32),
            pltpu.SemaphoreType.DMA,
        ],
    )
    def k(tab_hbm, idx_hbm, out_hbm, idx_v, rows_v, sem):
        wid = lax.axis_index("s") * _NC + lax.axis_index("c")
        pltpu.sync_copy(idx_hbm.at[wid], idx_v)
        base = wid * _RPW
        for g in range(_NCH):
            handles = []
            for j in range(_CH_STREAMS):
                handles.append(pltpu.async_copy(
                    tab_hbm.at[idx_v.at[g * _CH_STREAMS + j]],
                    rows_v.at[pl.ds(j * _IDXW, _IDXW)],
                    sem,
                ))
            for h in handles:
                h.wait()
            pltpu.sync_copy(
                rows_v, out_hbm.at[pl.ds(base + g * _CH_ROWS, _CH_ROWS)])

    return k(table, idx3d)


_TCOL = 4096
_TGRID = 124
_SPLIT = _TGRID * _TCOL
_VLAST = -(-1000000 // _TCOL) - 1


def _tr_block(xa_ref, xb_ref, o_ref):
    o_ref[:, :_D] = jnp.swapaxes(xa_ref[...], 0, 1)
    o_ref[:, _D:] = jnp.swapaxes(xb_ref[...], 0, 1)


def _tc_transpose(table_t):
    return pl.pallas_call(
        _tr_block,
        grid=(_TGRID,),
        in_specs=[
            pl.BlockSpec((_D, _TCOL), lambda i: (0, i)),
            pl.BlockSpec((_D, _TCOL),
                         lambda i: (0, jnp.minimum(i + _TGRID, _VLAST))),
        ],
        out_specs=pl.BlockSpec((_TCOL, 128), lambda i: (i, 0)),
        out_shape=jax.ShapeDtypeStruct((_SPLIT, 128), jnp.float32),
    )(table_t, table_t)


_SB = 8


def _ln_block(x_ref, pe_ref, g_ref, b_ref, o_ref):
    for si in range(_SB):
        xs = x_ref[si]
        t = jnp.concatenate(
            [jnp.swapaxes(xs[:, :_D], 0, 1),
             jnp.swapaxes(xs[:, _D:], 0, 1)], axis=1)
        x = t + pe_ref[si]
        m = jnp.mean(x, axis=0, keepdims=True)
        c = x - m
        v = jnp.mean(c * c, axis=0, keepdims=True)
        y = c * lax.rsqrt(v + _EPS)
        o_ref[si] = y * g_ref[0] + b_ref[0]


def _tc_layernorm(st3, pe_t, gamma_t, beta_t):
    grid = _S // _SB
    return pl.pallas_call(
        _ln_block,
        grid=(grid,),
        in_specs=[
            pl.BlockSpec((_SB, _B // 2, 128), lambda i: (i, 0, 0)),
            pl.BlockSpec((_SB, _D, 1), lambda i: (i, 0, 0)),
            pl.BlockSpec((1, _D, 1), lambda i: (0, 0, 0)),
            pl.BlockSpec((1, _D, 1), lambda i: (0, 0, 0)),
        ],
        out_specs=pl.BlockSpec((_SB, _D, _B), lambda i: (i, 0, 0)),
        out_shape=jax.ShapeDtypeStruct((_S, _D, _B), jnp.float32),
    )(st3, pe_t, gamma_t, beta_t)


def kernel(input_ids, table, gamma, beta, pos_enc):
    ids_p = (input_ids.T.astype(jnp.int32)
             .reshape(_S, 2, _B // 2).transpose(0, 2, 1).reshape(-1))
    gidx = jnp.where(ids_p < _SPLIT, ids_p * 2, (ids_p - _SPLIT) * 2 + 1)
    idx3d = gidx.reshape(_NW, _NSTREAM, _IDXW)
    lines = _tc_transpose(table.T)
    tab_lin = lines.reshape(-1).reshape(2 * _SPLIT, _D)
    staging = _sc_gather(tab_lin, idx3d)
    st3 = staging.reshape(-1).reshape(_S, _B // 2, 128)
    pe_t = pos_enc[0, :_S, :].reshape(_S, _D, 1)
    out_t = _tc_layernorm(st3, pe_t,
                          gamma.reshape(1, _D, 1), beta.reshape(1, _D, 1))
    return out_t.transpose(2, 0, 1)


# speedup vs baseline: 1.8374x; 1.1496x over previous
import functools

import jax
import jax.numpy as jnp
from jax import lax
from jax.experimental import pallas as pl
from jax.experimental.pallas import tpu as pltpu
from jax.experimental.pallas import tpu_sc as plsc

_NC = 2
_NS = 16
_NW = _NC * _NS

_B = 1024
_S = 200
_D = 64
_ROWS = _B * _S
_RPW = _ROWS // _NW
_IDXW = 128
_NSTREAM = _RPW // _IDXW
_CH_STREAMS = 10
_CH_ROWS = _CH_STREAMS * _IDXW
_NCH = _NSTREAM // _CH_STREAMS

_EPS = 1e-5


def _sc_gather(table, gidx_t):
    mesh = plsc.VectorSubcoreMesh(core_axis_name="c", subcore_axis_name="s")

    @functools.partial(
        pl.kernel,
        mesh=mesh,
        compiler_params=pltpu.CompilerParams(use_tc_tiling_on_sc=False,
                                             needs_layout_passes=False),
        out_type=jax.ShapeDtypeStruct((_ROWS, _D), jnp.float32),
        scratch_types=[
            pltpu.VMEM((7, _B), jnp.int32),
            pltpu.VMEM((_NSTREAM, _IDXW), jnp.int32),
            pltpu.VMEM((_CH_ROWS, _D), jnp.float32),
            pltpu.SemaphoreType.DMA,
        ],
    )
    def k(tab_hbm, idx_hbm, out_hbm, idx_raw, idx_v, rows_v, sem):
        wid = lax.axis_index("s") * _NC + lax.axis_index("c")
        s_lo = (wid * _NSTREAM) // 8
        pltpu.sync_copy(idx_hbm.at[pl.ds(s_lo, 7)], idx_raw)
        lane = lax.iota(jnp.int32, 16)
        zeros = jnp.zeros((16,), jnp.int32)
        for j in range(_NSTREAM):
            blk = wid * _NSTREAM + j
            row = zeros + (blk // 8 - s_lo)
            k0 = (blk % 8) * 64
            jrow = jnp.full((16,), j, jnp.int32)
            for g in range(4):
                col = k0 + g * 16 + lane
                for p in range(2):
                    vals = plsc.load_gather(idx_raw, [row, col + p * 512])
                    plsc.store_scatter(
                        idx_v, [jrow, g * 32 + lane * 2 + p], vals)
        base = wid * _RPW
        for g in range(_NCH):
            handles = []
            for j in range(_CH_STREAMS):
                handles.append(pltpu.async_copy(
                    tab_hbm.at[idx_v.at[g * _CH_STREAMS + j]],
                    rows_v.at[pl.ds(j * _IDXW, _IDXW)],
                    sem,
                ))
            for h in handles:
                h.wait()
            pltpu.sync_copy(
                rows_v, out_hbm.at[pl.ds(base + g * _CH_ROWS, _CH_ROWS)])

    return k(table, gidx_t)


_TCOL = 4096
_TGRID = 124
_SPLIT = _TGRID * _TCOL
_VLAST = -(-1000000 // _TCOL) - 1


def _tr_block(xa_ref, xb_ref, o_ref):
    o_ref[:, :_D] = jnp.swapaxes(xa_ref[...], 0, 1)
    o_ref[:, _D:] = jnp.swapaxes(xb_ref[...], 0, 1)


def _tc_transpose(table_t):
    return pl.pallas_call(
        _tr_block,
        grid=(_TGRID,),
        in_specs=[
            pl.BlockSpec((_D, _TCOL), lambda i: (0, i)),
            pl.BlockSpec((_D, _TCOL),
                         lambda i: (0, jnp.minimum(i + _TGRID, _VLAST))),
        ],
        out_specs=pl.BlockSpec((_TCOL, 128), lambda i: (i, 0)),
        out_shape=jax.ShapeDtypeStruct((_SPLIT, 128), jnp.float32),
    )(table_t, table_t)


_SB = 8


def _ln_block(x_ref, pe_ref, g_ref, b_ref, o_ref):
    for si in range(_SB):
        xs = x_ref[si]
        t = jnp.concatenate(
            [jnp.swapaxes(xs[:, :_D], 0, 1),
             jnp.swapaxes(xs[:, _D:], 0, 1)], axis=1)
        x = t + pe_ref[si]
        m = jnp.mean(x, axis=0, keepdims=True)
        c = x - m
        v = jnp.mean(c * c, axis=0, keepdims=True)
        y = c * lax.rsqrt(v + _EPS)
        o_ref[si] = y * g_ref[0] + b_ref[0]


def _tc_layernorm(st3, pe_t, gamma_t, beta_t):
    grid = _S // _SB
    return pl.pallas_call(
        _ln_block,
        grid=(grid,),
        in_specs=[
            pl.BlockSpec((_SB, _B // 2, 128), lambda i: (i, 0, 0)),
            pl.BlockSpec((_SB, _D, 1), lambda i: (i, 0, 0)),
            pl.BlockSpec((1, _D, 1), lambda i: (0, 0, 0)),
            pl.BlockSpec((1, _D, 1), lambda i: (0, 0, 0)),
        ],
        out_specs=pl.BlockSpec((_SB, _D, _B), lambda i: (i, 0, 0)),
        out_shape=jax.ShapeDtypeStruct((_S, _D, _B), jnp.float32),
    )(st3, pe_t, gamma_t, beta_t)


def kernel(input_ids, table, gamma, beta, pos_enc):
    ids_t = input_ids.T.astype(jnp.int32)
    gidx_t = jnp.where(ids_t < _SPLIT, ids_t * 2, (ids_t - _SPLIT) * 2 + 1)
    lines = _tc_transpose(table.T)
    tab_lin = lines.reshape(-1).reshape(2 * _SPLIT, _D)
    staging = _sc_gather(tab_lin, gidx_t)
    st3 = staging.reshape(-1).reshape(_S, _B // 2, 128)
    pe_t = pos_enc[0, :_S, :].reshape(_S, _D, 1)
    out_t = _tc_layernorm(st3, pe_t,
                          gamma.reshape(1, _D, 1), beta.reshape(1, _D, 1))
    return out_t.transpose(2, 0, 1)


# speedup vs baseline: 2.0121x; 1.0951x over previous
import functools

import jax
import jax.numpy as jnp
from jax import lax
from jax.experimental import pallas as pl
from jax.experimental.pallas import tpu as pltpu
from jax.experimental.pallas import tpu_sc as plsc

_NC = 2
_NS = 16
_NW = _NC * _NS

_B = 1024
_S = 200
_D = 64
_ROWS = _B * _S
_RPW = _ROWS // _NW
_IDXW = 128
_NSTREAM = _RPW // _IDXW
_CH_STREAMS = 10
_CH_ROWS = _CH_STREAMS * _IDXW
_NCH = _NSTREAM // _CH_STREAMS

_EPS = 1e-5


def _sc_gather(table, gidx_t):
    mesh = plsc.VectorSubcoreMesh(core_axis_name="c", subcore_axis_name="s")

    @functools.partial(
        pl.kernel,
        mesh=mesh,
        compiler_params=pltpu.CompilerParams(use_tc_tiling_on_sc=False,
                                             needs_layout_passes=False),
        out_type=jax.ShapeDtypeStruct((_ROWS, _D), jnp.float32),
        scratch_types=[
            pltpu.VMEM((7, _B), jnp.int32),
            pltpu.VMEM((_NSTREAM, _IDXW), jnp.int32),
            pltpu.VMEM((_CH_ROWS, _D), jnp.float32),
            pltpu.SemaphoreType.DMA,
        ],
    )
    def k(tab_hbm, idx_hbm, out_hbm, idx_raw, idx_v, rows_v, sem):
        wid = lax.axis_index("s") * _NC + lax.axis_index("c")
        s_lo = (wid * _NSTREAM) // 8
        pltpu.sync_copy(idx_hbm.at[pl.ds(s_lo, 7)], idx_raw)
        lane = lax.iota(jnp.int32, 16)
        zeros = jnp.zeros((16,), jnp.int32)
        for j in range(_NSTREAM):
            blk = wid * _NSTREAM + j
            row = zeros + (blk // 8 - s_lo)
            k0 = (blk % 8) * 64
            jrow = jnp.full((16,), j, jnp.int32)
            for g in range(4):
                col = k0 + g * 16 + lane
                for p in range(2):
                    vals = plsc.load_gather(idx_raw, [row, col + p * 512])
                    plsc.store_scatter(
                        idx_v, [jrow, g * 32 + lane * 2 + p], vals)
        base = wid * _RPW
        for g in range(_NCH):
            handles = []
            for j in range(_CH_STREAMS):
                handles.append(pltpu.async_copy(
                    tab_hbm.at[idx_v.at[g * _CH_STREAMS + j]],
                    rows_v.at[pl.ds(j * _IDXW, _IDXW)],
                    sem,
                ))
            for h in handles:
                h.wait()
            pltpu.sync_copy(
                rows_v, out_hbm.at[pl.ds(base + g * _CH_ROWS, _CH_ROWS)])

    return k(table, gidx_t)


_TCOL = 8192
_TGRID = 62
_SPLIT = _TGRID * _TCOL
_VLAST = -(-1000000 // _TCOL) - 1


def _tr_block(xa_ref, xb_ref, o_ref):
    o_ref[:, :_D] = jnp.swapaxes(xa_ref[...], 0, 1)
    o_ref[:, _D:] = jnp.swapaxes(xb_ref[...], 0, 1)


def _tc_transpose(table_t):
    return pl.pallas_call(
        _tr_block,
        grid=(_TGRID,),
        in_specs=[
            pl.BlockSpec((_D, _TCOL), lambda i: (0, i)),
            pl.BlockSpec((_D, _TCOL),
                         lambda i: (0, jnp.minimum(i + _TGRID, _VLAST))),
        ],
        out_specs=pl.BlockSpec((_TCOL, 128), lambda i: (i, 0)),
        out_shape=jax.ShapeDtypeStruct((_SPLIT, 128), jnp.float32),
    )(table_t, table_t)


_SB = 8


def _ln_block(x_ref, pe_ref, g_ref, b_ref, o_ref):
    for si in range(_SB):
        xs = x_ref[si]
        t = jnp.concatenate(
            [jnp.swapaxes(xs[:, :_D], 0, 1),
             jnp.swapaxes(xs[:, _D:], 0, 1)], axis=1)
        x = t + pe_ref[si]
        m = jnp.mean(x, axis=0, keepdims=True)
        c = x - m
        v = jnp.mean(c * c, axis=0, keepdims=True)
        y = c * lax.rsqrt(v + _EPS)
        o_ref[si] = y * g_ref[0] + b_ref[0]


def _tc_layernorm(st3, pe_t, gamma_t, beta_t):
    grid = _S // _SB
    return pl.pallas_call(
        _ln_block,
        grid=(grid,),
        in_specs=[
            pl.BlockSpec((_SB, _B // 2, 128), lambda i: (i, 0, 0)),
            pl.BlockSpec((_SB, _D, 1), lambda i: (i, 0, 0)),
            pl.BlockSpec((1, _D, 1), lambda i: (0, 0, 0)),
            pl.BlockSpec((1, _D, 1), lambda i: (0, 0, 0)),
        ],
        out_specs=pl.BlockSpec((_SB, _D, _B), lambda i: (i, 0, 0)),
        out_shape=jax.ShapeDtypeStruct((_S, _D, _B), jnp.float32),
    )(st3, pe_t, gamma_t, beta_t)


def kernel(input_ids, table, gamma, beta, pos_enc):
    ids_t = input_ids.T.astype(jnp.int32)
    gidx_t = jnp.where(ids_t < _SPLIT, ids_t * 2, (ids_t - _SPLIT) * 2 + 1)
    lines = _tc_transpose(table.T)
    tab_lin = lines.reshape(-1).reshape(2 * _SPLIT, _D)
    staging = _sc_gather(tab_lin, gidx_t)
    st3 = staging.reshape(-1).reshape(_S, _B // 2, 128)
    pe_t = pos_enc[0, :_S, :].reshape(_S, _D, 1)
    out_t = _tc_layernorm(st3, pe_t,
                          gamma.reshape(1, _D, 1), beta.reshape(1, _D, 1))
    return out_t.transpose(2, 0, 1)


# speedup vs baseline: 2.0901x; 1.0388x over previous
import functools

import jax
import jax.numpy as jnp
from jax import lax
from jax.experimental import pallas as pl
from jax.experimental.pallas import tpu as pltpu
from jax.experimental.pallas import tpu_sc as plsc

_NC = 2
_NS = 16
_NW = _NC * _NS

_B = 1024
_S = 200
_D = 64
_ROWS = _B * _S
_RPW = _ROWS // _NW
_IDXW = 128
_NSTREAM = _RPW // _IDXW
_CH_STREAMS = 10
_CH_ROWS = _CH_STREAMS * _IDXW
_NCH = _NSTREAM // _CH_STREAMS

_EPS = 1e-5


def _sc_gather(table, gidx_t):
    mesh = plsc.VectorSubcoreMesh(core_axis_name="c", subcore_axis_name="s")

    @functools.partial(
        pl.kernel,
        mesh=mesh,
        compiler_params=pltpu.CompilerParams(use_tc_tiling_on_sc=False,
                                             needs_layout_passes=False),
        out_type=jax.ShapeDtypeStruct((_ROWS, _D), jnp.float32),
        scratch_types=[
            pltpu.VMEM((7, _B), jnp.int32),
            pltpu.VMEM((_NSTREAM, _IDXW), jnp.int32),
            pltpu.VMEM((_CH_ROWS, _D), jnp.float32),
            pltpu.SemaphoreType.DMA,
        ],
    )
    def k(tab_hbm, idx_hbm, out_hbm, idx_raw, idx_v, rows_v, sem):
        wid = lax.axis_index("s") * _NC + lax.axis_index("c")
        s_lo = (wid * _NSTREAM) // 8
        pltpu.sync_copy(idx_hbm.at[pl.ds(s_lo, 7)], idx_raw)
        lane = lax.iota(jnp.int32, 16)
        zeros = jnp.zeros((16,), jnp.int32)
        for j in range(_NSTREAM):
            blk = wid * _NSTREAM + j
            row = zeros + (blk // 8 - s_lo)
            k0 = (blk % 8) * 64
            jrow = jnp.full((16,), j, jnp.int32)
            for g in range(4):
                col = k0 + g * 16 + lane
                for p in range(2):
                    vals = plsc.load_gather(idx_raw, [row, col + p * 512])
                    plsc.store_scatter(
                        idx_v, [jrow, g * 32 + lane * 2 + p], vals)
        base = wid * _RPW
        for g in range(_NCH):
            handles = []
            for j in range(_CH_STREAMS):
                handles.append(pltpu.async_copy(
                    tab_hbm.at[idx_v.at[g * _CH_STREAMS + j]],
                    rows_v.at[pl.ds(j * _IDXW, _IDXW)],
                    sem,
                ))
            for h in handles:
                h.wait()
            pltpu.sync_copy(
                rows_v, out_hbm.at[pl.ds(base + g * _CH_ROWS, _CH_ROWS)])

    return k(table, gidx_t)


_TCOL = 16384
_TGRID = 31
_SPLIT = _TGRID * _TCOL
_VLAST = -(-1000000 // _TCOL) - 1


def _tr_block(xa_ref, xb_ref, o_ref):
    o_ref[:, :_D] = jnp.swapaxes(xa_ref[...], 0, 1)
    o_ref[:, _D:] = jnp.swapaxes(xb_ref[...], 0, 1)


def _tc_transpose(table_t):
    return pl.pallas_call(
        _tr_block,
        grid=(_TGRID,),
        in_specs=[
            pl.BlockSpec((_D, _TCOL), lambda i: (0, i)),
            pl.BlockSpec((_D, _TCOL),
                         lambda i: (0, jnp.minimum(i + _TGRID, _VLAST))),
        ],
        out_specs=pl.BlockSpec((_TCOL, 128), lambda i: (i, 0)),
        out_shape=jax.ShapeDtypeStruct((_SPLIT, 128), jnp.float32),
    )(table_t, table_t)


_SB = 8


def _ln_block(x_ref, pe_ref, g_ref, b_ref, o_ref):
    for si in range(_SB):
        xs = x_ref[si]
        t = jnp.concatenate(
            [jnp.swapaxes(xs[:, :_D], 0, 1),
             jnp.swapaxes(xs[:, _D:], 0, 1)], axis=1)
        x = t + pe_ref[si]
        m = jnp.mean(x, axis=0, keepdims=True)
        c = x - m
        v = jnp.mean(c * c, axis=0, keepdims=True)
        y = c * lax.rsqrt(v + _EPS)
        o_ref[si] = y * g_ref[0] + b_ref[0]


def _tc_layernorm(st3, pe_t, gamma_t, beta_t):
    grid = _S // _SB
    return pl.pallas_call(
        _ln_block,
        grid=(grid,),
        in_specs=[
            pl.BlockSpec((_SB, _B // 2, 128), lambda i: (i, 0, 0)),
            pl.BlockSpec((_SB, _D, 1), lambda i: (i, 0, 0)),
            pl.BlockSpec((1, _D, 1), lambda i: (0, 0, 0)),
            pl.BlockSpec((1, _D, 1), lambda i: (0, 0, 0)),
        ],
        out_specs=pl.BlockSpec((_SB, _D, _B), lambda i: (i, 0, 0)),
        out_shape=jax.ShapeDtypeStruct((_S, _D, _B), jnp.float32),
    )(st3, pe_t, gamma_t, beta_t)


def kernel(input_ids, table, gamma, beta, pos_enc):
    ids_t = input_ids.T.astype(jnp.int32)
    gidx_t = jnp.where(ids_t < _SPLIT, ids_t * 2, (ids_t - _SPLIT) * 2 + 1)
    lines = _tc_transpose(table.T)
    tab_lin = lines.reshape(-1).reshape(2 * _SPLIT, _D)
    staging = _sc_gather(tab_lin, gidx_t)
    st3 = staging.reshape(-1).reshape(_S, _B // 2, 128)
    pe_t = pos_enc[0, :_S, :].reshape(_S, _D, 1)
    out_t = _tc_layernorm(st3, pe_t,
                          gamma.reshape(1, _D, 1), beta.reshape(1, _D, 1))
    return out_t.transpose(2, 0, 1)
